# baseline (speedup 1.0000x reference)
import functools

import jax
import jax.numpy as jnp
from jax import lax
from jax.experimental import pallas as pl
from jax.experimental.pallas import tpu as pltpu
from jax.experimental.pallas import tpu_sc as plsc

VOCAB = 1000000
DIM = 32
NUM_LABELS = 10
BATCH = 4096
SEQ = 200

NC = 2
NS = 16
L = 16
NW = NC * NS
BPW = BATCH // NW
FLAT = BPW * SEQ
C1, C2 = 128, SEQ - 128

_mesh = plsc.VectorSubcoreMesh(core_axis_name="c", subcore_axis_name="s")


@functools.partial(
    pl.kernel,
    mesh=_mesh,
    out_type=jax.ShapeDtypeStruct((BATCH, DIM), jnp.float32),
    compiler_params=pltpu.CompilerParams(use_tc_tiling_on_sc=False),
    scratch_types=[
        pltpu.VMEM((FLAT + L,), jnp.int32),
        pltpu.VMEM((FLAT + L,), jnp.int32),
        pltpu.VMEM((SEQ, DIM), jnp.float32),
        pltpu.VMEM((SEQ, DIM), jnp.float32),
        pltpu.VMEM((BPW, DIM), jnp.float32),
        pltpu.VMEM((1, DIM), jnp.float32),
        pltpu.SemaphoreType.DMA,
        pltpu.SemaphoreType.DMA,
    ],
)
def _sc_pool(ids_hbm, mask_hbm, table_hbm, out_hbm,
             ids_v, mask_v, rows_a, rows_b, pooled_v, t0_v, sem_a, sem_b):
    wid = lax.axis_index("s") * NC + lax.axis_index("c")
    base = wid * FLAT

    pltpu.sync_copy(ids_hbm.at[pl.ds(base, FLAT)], ids_v.at[pl.ds(0, FLAT)])
    pltpu.sync_copy(mask_hbm.at[pl.ds(base, FLAT)], mask_v.at[pl.ds(0, FLAT)])
    pltpu.sync_copy(table_hbm.at[pl.ds(0, 1), :], t0_v)

    ids_v[pl.ds(FLAT, L)] = jnp.zeros((L,), jnp.int32)
    mask_v[pl.ds(FLAT, L)] = jnp.zeros((L,), jnp.int32)

    def _mask_ids(i, carry):
        sl = pl.ds(i * L, L)
        ids_v[sl] = ids_v[sl] * mask_v[sl]
        return carry

    lax.fori_loop(0, FLAT // L, _mask_ids, 0)

    t0_lo = t0_v[0, pl.ds(0, L)]
    t0_hi = t0_v[0, pl.ds(L, L)]

    iot = lax.iota(jnp.int32, L)
    thresh = jnp.full((L,), SEQ % L, jnp.int32)
    ones_i = jnp.full((L,), 1, jnp.int32)
    zeros_i = jnp.full((L,), 0, jnp.int32)
    lane = jnp.where(iot < thresh, ones_i, zeros_i)
    seq_f = jnp.full((L,), float(SEQ), jnp.float32)
    one_f = jnp.full((L,), 1.0, jnp.float32)

    def _start(b, rows, sem):
        off = b * SEQ
        cp1 = pltpu.async_copy(table_hbm.at[ids_v.at[pl.ds(off, C1)]],
                               rows.at[pl.ds(0, C1), :], sem)
        cp2 = pltpu.async_copy(table_hbm.at[ids_v.at[pl.ds(off + C1, C2)]],
                               rows.at[pl.ds(C1, C2), :], sem)
        return cp1, cp2

    def _finish(b, rows, cps):
        cps[0].wait()
        cps[1].wait()
        off = b * SEQ

        def _cnt(j, c):
            return c + mask_v[pl.ds(off + j * L, L)]

        cvec = lax.fori_loop(0, SEQ // L, _cnt, zeros_i)
        tail = mask_v[pl.ds(off + (SEQ // L) * L, L)]
        cvec = cvec + tail * lane
        for sh in (8, 4, 2, 1):
            perm = iot ^ jnp.full((L,), sh, jnp.int32)
            cvec = cvec + cvec.at[perm].get(mode="promise_in_bounds")

        def _acc(s, carry):
            a0, a1 = carry
            return (a0 + rows[s, pl.ds(0, L)], a1 + rows[s, pl.ds(L, L)])

        zero = jnp.zeros((L,), jnp.float32)
        a0, a1 = lax.fori_loop(0, SEQ, _acc, (zero, zero))

        cnt_f = cvec.astype(jnp.float32)
        dummy = seq_f - cnt_f
        inv = one_f / jnp.maximum(cnt_f, one_f)
        pooled_v[b, pl.ds(0, L)] = (a0 - dummy * t0_lo) * inv
        pooled_v[b, pl.ds(L, L)] = (a1 - dummy * t0_hi) * inv

    cps0 = _start(0, rows_a, sem_a)

    def _pair(g, carry):
        b0 = g * 2
        cps_b = _start(b0 + 1, rows_b, sem_b)
        _finish(b0, rows_a, cps0)

        @pl.when(b0 + 2 < BPW)
        def _():
            _start(b0 + 2, rows_a, sem_a)

        _finish(b0 + 1, rows_b, cps_b)
        return carry

    lax.fori_loop(0, BPW // 2, _pair, 0)

    pltpu.sync_copy(pooled_v, out_hbm.at[pl.ds(wid * BPW, BPW), :])


def _head_body(p_ref, w_ref, b_ref, o_ref):
    o_ref[...] = (
        jnp.dot(p_ref[...], w_ref[...], preferred_element_type=jnp.float32)
        + b_ref[...]
    )


_head = pl.pallas_call(
    _head_body,
    out_shape=jax.ShapeDtypeStruct((BATCH, NUM_LABELS), jnp.float32),
)


def kernel(input_ids, attention_mask, table, W, b):
    ids_flat = input_ids.reshape(-1).astype(jnp.int32)
    mask_flat = attention_mask.reshape(-1).astype(jnp.int32)
    pooled = _sc_pool(ids_flat, mask_flat, table)
    logits = _head(pooled, W, b.reshape(1, NUM_LABELS))
    return (logits, pooled)


# speedup vs baseline: 1.0004x; 1.0004x over previous
import functools

import jax
import jax.numpy as jnp
from jax import lax
from jax.experimental import pallas as pl
from jax.experimental.pallas import tpu as pltpu
from jax.experimental.pallas import tpu_sc as plsc

VOCAB = 1000000
DIM = 32
NUM_LABELS = 10
BATCH = 4096
SEQ = 200

NC = 2
NS = 16
L = 16
NW = NC * NS
BPW = BATCH // NW
FLAT = BPW * SEQ
C1, C2 = 128, SEQ - 128

_mesh = plsc.VectorSubcoreMesh(core_axis_name="c", subcore_axis_name="s")


@functools.partial(
    pl.kernel,
    mesh=_mesh,
    out_type=jax.ShapeDtypeStruct((BATCH, DIM), jnp.float32),
    compiler_params=pltpu.CompilerParams(use_tc_tiling_on_sc=False),
    scratch_types=[
        pltpu.VMEM((FLAT + L,), jnp.int32),
        pltpu.VMEM((FLAT + L,), jnp.int32),
        pltpu.VMEM((SEQ, DIM), jnp.float32),
        pltpu.VMEM((SEQ, DIM), jnp.float32),
        pltpu.VMEM((BPW, DIM), jnp.float32),
        pltpu.VMEM((1, DIM), jnp.float32),
        pltpu.SemaphoreType.DMA,
        pltpu.SemaphoreType.DMA,
    ],
)
def _sc_pool(ids_hbm, mask_hbm, table_hbm, out_hbm,
             ids_v, mask_v, rows_a, rows_b, pooled_v, t0_v, sem_a, sem_b):
    wid = lax.axis_index("s") * NC + lax.axis_index("c")
    base = wid * FLAT

    pltpu.sync_copy(ids_hbm.at[pl.ds(base, FLAT)], ids_v.at[pl.ds(0, FLAT)])
    pltpu.sync_copy(mask_hbm.at[pl.ds(base, FLAT)], mask_v.at[pl.ds(0, FLAT)])
    pltpu.sync_copy(table_hbm.at[pl.ds(0, 1), :], t0_v)

    ids_v[pl.ds(FLAT, L)] = jnp.zeros((L,), jnp.int32)
    mask_v[pl.ds(FLAT, L)] = jnp.zeros((L,), jnp.int32)

    MU = 8

    def _mask_ids(i, carry):
        for k in range(MU):
            sl = pl.ds((i * MU + k) * L, L)
            ids_v[sl] = ids_v[sl] * mask_v[sl]
        return carry

    lax.fori_loop(0, FLAT // (L * MU), _mask_ids, 0)

    t0_lo = t0_v[0, pl.ds(0, L)]
    t0_hi = t0_v[0, pl.ds(L, L)]

    iot = lax.iota(jnp.int32, L)
    thresh = jnp.full((L,), SEQ % L, jnp.int32)
    ones_i = jnp.full((L,), 1, jnp.int32)
    zeros_i = jnp.full((L,), 0, jnp.int32)
    lane = jnp.where(iot < thresh, ones_i, zeros_i)
    seq_f = jnp.full((L,), float(SEQ), jnp.float32)
    one_f = jnp.full((L,), 1.0, jnp.float32)

    def _start(b, rows, sem):
        off = b * SEQ
        cp1 = pltpu.async_copy(table_hbm.at[ids_v.at[pl.ds(off, C1)]],
                               rows.at[pl.ds(0, C1), :], sem)
        cp2 = pltpu.async_copy(table_hbm.at[ids_v.at[pl.ds(off + C1, C2)]],
                               rows.at[pl.ds(C1, C2), :], sem)
        return cp1, cp2

    def _finish(b, rows, cps):
        off = b * SEQ

        cvec = mask_v[pl.ds(off + (SEQ // L) * L, L)] * lane
        for j in range(SEQ // L):
            cvec = cvec + mask_v[pl.ds(off + j * L, L)]
        for sh in (8, 4, 2, 1):
            perm = iot ^ jnp.full((L,), sh, jnp.int32)
            cvec = cvec + cvec.at[perm].get(mode="promise_in_bounds")

        cps[0].wait()
        cps[1].wait()

        AU = 8
        zero = jnp.zeros((L,), jnp.float32)

        def _acc(i, carry):
            a0, a1, a2, a3 = carry
            s0 = i * AU
            for k in range(AU):
                lo = rows[s0 + k, pl.ds(0, L)]
                hi = rows[s0 + k, pl.ds(L, L)]
                if k % 2 == 0:
                    a0 = a0 + lo
                    a1 = a1 + hi
                else:
                    a2 = a2 + lo
                    a3 = a3 + hi
            return (a0, a1, a2, a3)

        a0, a1, a2, a3 = lax.fori_loop(0, SEQ // AU, _acc,
                                       (zero, zero, zero, zero))
        a0 = a0 + a2
        a1 = a1 + a3

        cnt_f = cvec.astype(jnp.float32)
        dummy = seq_f - cnt_f
        inv = one_f / jnp.maximum(cnt_f, one_f)
        pooled_v[b, pl.ds(0, L)] = (a0 - dummy * t0_lo) * inv
        pooled_v[b, pl.ds(L, L)] = (a1 - dummy * t0_hi) * inv

    cps0 = _start(0, rows_a, sem_a)

    def _pair(g, carry):
        b0 = g * 2
        cps_b = _start(b0 + 1, rows_b, sem_b)
        _finish(b0, rows_a, cps0)

        @pl.when(b0 + 2 < BPW)
        def _():
            _start(b0 + 2, rows_a, sem_a)

        _finish(b0 + 1, rows_b, cps_b)
        return carry

    lax.fori_loop(0, BPW // 2, _pair, 0)

    pltpu.sync_copy(pooled_v, out_hbm.at[pl.ds(wid * BPW, BPW), :])


def _head_body(p_ref, w_ref, b_ref, o_ref):
    o_ref[...] = (
        jnp.dot(p_ref[...], w_ref[...], preferred_element_type=jnp.float32)
        + b_ref[...]
    )


_head = pl.pallas_call(
    _head_body,
    out_shape=jax.ShapeDtypeStruct((BATCH, NUM_LABELS), jnp.float32),
)


def kernel(input_ids, attention_mask, table, W, b):
    ids_flat = input_ids.reshape(-1).astype(jnp.int32)
    mask_flat = attention_mask.reshape(-1).astype(jnp.int32)
    pooled = _sc_pool(ids_flat, mask_flat, table)
    logits = _head(pooled, W, b.reshape(1, NUM_LABELS))
    return (logits, pooled)
